# Initial kernel scaffold; baseline (speedup 1.0000x reference)
import jax
import jax.numpy as jnp
from jax.experimental import pallas as pl


def kernel(dag_x, dag_edge_index, res_x, res_edge_index, dag_f1_Wl, dag_f1_Wr, dag_f1_b, dag_b1_Wl, dag_b1_Wr, dag_b1_b, dag_f2_Wl, dag_f2_Wr, dag_f2_b, dag_b2_Wl, dag_b2_Wr, dag_b2_b, dag_bn1_g, dag_bn1_b, dag_bn2_g, dag_bn2_b, res_c1_Wl, res_c1_Wr, res_c1_b, res_c2_Wl, res_c2_Wr, res_c2_b, res_bn1_g, res_bn1_b, res_bn2_g, res_bn2_b, joint_W, joint_b):
    raise NotImplementedError("write your pallas kernel here")


# speedup vs baseline: 9.8382x; 9.8382x over previous
import jax
import jax.numpy as jnp
from jax import lax
from jax.experimental import pallas as pl
from jax.experimental.pallas import tpu as pltpu
from jax.experimental.pallas import tpu_sc as plsc

N_D, E_D = 50000, 800000
N_R, E_R = 10000, 320000
H = 64
BR = 1024
NB_D, NB_R = 49, 10
NDP = NB_D * BR
NRP = NB_R * BR
CH = 128
KB = 8
NTILES = 16
DCHP = 6400
RCHP = 2560

_mesh = lambda: plsc.VectorSubcoreMesh(core_axis_name="c", subcore_axis_name="s",
                                       num_cores=2, num_subcores=16)
_SC_PARAMS = pltpu.CompilerParams(use_tc_tiling_on_sc=False)


def _zero_fill(slab_v, srows):
    z = jnp.zeros((16,), jnp.float32)
    width = slab_v.shape[1]

    def zb(i, _):
        for k in range(8):
            for c0 in range(0, width, 16):
                slab_v[i * 8 + k, c0:c0 + 16] = z
        return _

    lax.fori_loop(0, srows // 8, zb, None)


def _sc_seg16(n_pad, blocks_per_tile, fwd):
    rows_per_tile = n_pad // NTILES
    srows = rows_per_tile // 2
    gi, si = (0, 1) if fwd else (1, 0)
    scratch = [
        pltpu.VMEM((KB, 2, CH), jnp.int32),
        pltpu.VMEM((KB, CH, 16), jnp.float32),
        pltpu.VMEM((srows, 16), jnp.float32),
        pltpu.VMEM_SHARED((n_pad, 16), jnp.float32),
        pltpu.SemaphoreType.DMA, pltpu.SemaphoreType.DMA,
    ]

    def body(x_hbm, e_hbm, out, idx_v, rows_v, slab_v, acc, semg, sems):
        c = lax.axis_index("c")
        s = lax.axis_index("s")
        wid = s * 2 + c
        _zero_fill(slab_v, srows)
        row0 = s * rows_per_tile
        for h in range(2):
            pltpu.sync_copy(slab_v, acc.at[pl.ds(row0 + h * srows, srows), :])
        plsc.subcore_barrier()
        base = wid * blocks_per_tile * KB

        def eb(t, _):
            cb = base + t * KB
            pltpu.sync_copy(e_hbm.at[pl.ds(cb, KB)], idx_v)
            ds = [pltpu.async_copy(x_hbm.at[idx_v.at[k, gi]], rows_v.at[k], semg)
                  for k in range(KB)]
            for d in ds:
                d.wait()
            ds2 = [pltpu.async_copy(rows_v.at[k], acc.at[idx_v.at[k, si]], sems,
                                    add=True) for k in range(KB)]
            for d in ds2:
                d.wait()
            return _

        lax.fori_loop(0, blocks_per_tile, eb, None)
        plsc.subcore_barrier()
        off = c * n_pad + row0
        for h in range(2):
            pltpu.sync_copy(acc.at[pl.ds(row0 + h * srows, srows), :], slab_v)
            pltpu.sync_copy(slab_v, out.at[pl.ds(off + h * srows, srows), :])

    return pl.kernel(body, out_type=jax.ShapeDtypeStruct((2 * n_pad, 16),
                                                         jnp.float32),
                     mesh=_mesh(), scratch_types=scratch,
                     compiler_params=_SC_PARAMS)


def _sc_seg_strip(n_pad, blocks_per_tile, fwd, width):
    rows_per_tile = n_pad // NTILES
    srows = rows_per_tile // 2
    gi, si = (0, 1) if fwd else (1, 0)
    scratch = [
        pltpu.VMEM((KB, 2, CH), jnp.int32),
        pltpu.VMEM((KB, CH, width), jnp.float32),
        pltpu.VMEM((srows, width), jnp.float32),
        pltpu.VMEM_SHARED((n_pad, width), jnp.float32),
        pltpu.SemaphoreType.DMA, pltpu.SemaphoreType.DMA,
    ]

    def body(ta_hbm, tb_hbm, e_hbm, out, idx_v, rows_v, slab_v, acc, semg, sems):
        c = lax.axis_index("c")
        s = lax.axis_index("s")
        _zero_fill(slab_v, srows)
        row0 = s * rows_per_tile
        for h in range(2):
            pltpu.sync_copy(slab_v, acc.at[pl.ds(row0 + h * srows, srows), :])
        plsc.subcore_barrier()
        base = s * blocks_per_tile * KB

        def eb(t, _):
            cb = base + t * KB
            pltpu.sync_copy(e_hbm.at[pl.ds(cb, KB)], idx_v)

            @pl.when(c == 0)
            def _g0():
                ds = [pltpu.async_copy(ta_hbm.at[idx_v.at[k, gi]], rows_v.at[k],
                                       semg) for k in range(KB)]
                for d in ds:
                    d.wait()

            @pl.when(c == 1)
            def _g1():
                ds = [pltpu.async_copy(tb_hbm.at[idx_v.at[k, gi]], rows_v.at[k],
                                       semg) for k in range(KB)]
                for d in ds:
                    d.wait()

            ds2 = [pltpu.async_copy(rows_v.at[k], acc.at[idx_v.at[k, si]], sems,
                                    add=True) for k in range(KB)]
            for d in ds2:
                d.wait()
            return _

        lax.fori_loop(0, blocks_per_tile, eb, None)
        plsc.subcore_barrier()
        off = c * n_pad + row0
        for h in range(2):
            pltpu.sync_copy(acc.at[pl.ds(row0 + h * srows, srows), :], slab_v)
            pltpu.sync_copy(slab_v, out.at[pl.ds(off + h * srows, srows), :])

    return pl.kernel(body, out_type=jax.ShapeDtypeStruct((2 * n_pad, width),
                                                         jnp.float32),
                     mesh=_mesh(), scratch_types=scratch,
                     compiler_params=_SC_PARAMS)


def _rowmask(i, n_nodes):
    rows = i * BR + lax.broadcasted_iota(jnp.int32, (BR, 1), 0)
    return rows < n_nodes


def _stats_accum(i, st_out, p):
    st = jnp.concatenate([jnp.sum(p, 0, keepdims=True),
                          jnp.sum(p * p, 0, keepdims=True)], 0)

    @pl.when(i == 0)
    def _():
        st_out[...] = st

    @pl.when(i > 0)
    def _():
        st_out[...] = st_out[...] + st


def _tc_pre1(n_nodes, nb, deg_col, two_dir):
    def kern(*args):
        if two_dir:
            (sf0, sf1, sb0, sb1, x, wlf, wlb, wr, b, p_out, st_out) = args
        else:
            (sf0, sf1, x, wlf, wr, b, p_out, st_out) = args
        i = pl.program_id(0)
        sfb = sf0[...] + sf1[...]
        mf = sfb / jnp.maximum(sfb[:, deg_col:deg_col + 1], 1.0)
        p = jnp.dot(mf, wlf[...], preferred_element_type=jnp.float32)
        if two_dir:
            sbb = sb0[...] + sb1[...]
            mb = sbb / jnp.maximum(sbb[:, deg_col:deg_col + 1], 1.0)
            p = p + jnp.dot(mb, wlb[...], preferred_element_type=jnp.float32)
        p = p + jnp.dot(x[...], wr[...], preferred_element_type=jnp.float32)
        p = p + b[...]
        p = jnp.where(_rowmask(i, n_nodes), p, 0.0)
        p_out[...] = p
        _stats_accum(i, st_out, p)

    n_pad = nb * BR
    half = lambda j: pl.BlockSpec((BR, 16), lambda i, j=j: (i + j * nb, 0))
    full16 = pl.BlockSpec((16, 64), lambda i: (0, 0))
    in_specs = [half(0), half(1)]
    if two_dir:
        in_specs += [half(0), half(1)]
    in_specs += [pl.BlockSpec((BR, 16), lambda i: (i, 0)), full16]
    if two_dir:
        in_specs += [full16]
    in_specs += [full16, pl.BlockSpec((1, 64), lambda i: (0, 0))]
    return pl.pallas_call(
        kern, grid=(nb,), in_specs=in_specs,
        out_specs=[pl.BlockSpec((BR, 64), lambda i: (i, 0)),
                   pl.BlockSpec((2, 64), lambda i: (0, 0))],
        out_shape=[jax.ShapeDtypeStruct((n_pad, 64), jnp.float32),
                   jax.ShapeDtypeStruct((2, 64), jnp.float32)])


def _tc_pre2(n_nodes, nb, deg_col, two_dir, nsplit):
    width = 64 // nsplit

    def kern(*args):
        args = list(args)
        s2f = [args.pop(0) for _ in range(nsplit)]
        s2b = [args.pop(0) for _ in range(nsplit)] if two_dir else None
        hs = [args.pop(0) for _ in range(nsplit)]
        sf0, sf1 = args.pop(0), args.pop(0)
        sb = (args.pop(0), args.pop(0)) if two_dir else None
        if two_dir:
            wlf, wlb, wr, b, p_out, st_out = args
        else:
            wlf, wr, b, p_out, st_out = args
        i = pl.program_id(0)
        degf = jnp.maximum(sf0[:, deg_col:deg_col + 1]
                           + sf1[:, deg_col:deg_col + 1], 1.0)
        m2f = jnp.concatenate([r[...] for r in s2f], axis=1) / degf
        p = jnp.dot(m2f, wlf[...], preferred_element_type=jnp.float32)
        if two_dir:
            degb = jnp.maximum(sb[0][:, deg_col:deg_col + 1]
                               + sb[1][:, deg_col:deg_col + 1], 1.0)
            m2b = jnp.concatenate([r[...] for r in s2b], axis=1) / degb
            p = p + jnp.dot(m2b, wlb[...], preferred_element_type=jnp.float32)
        hcat = jnp.concatenate([r[...] for r in hs], axis=1)
        p = p + jnp.dot(hcat, wr[...], preferred_element_type=jnp.float32)
        p = p + b[...]
        p = jnp.where(_rowmask(i, n_nodes), p, 0.0)
        p_out[...] = p
        _stats_accum(i, st_out, p)

    n_pad = nb * BR
    strip = lambda j: pl.BlockSpec((BR, width), lambda i, j=j: (i + j * nb, 0))
    s16 = lambda j: pl.BlockSpec((BR, 16), lambda i, j=j: (i + j * nb, 0))
    hstrip = pl.BlockSpec((BR, width), lambda i: (i, 0))
    full64 = pl.BlockSpec((64, 64), lambda i: (0, 0))
    in_specs = [strip(j % 2) for j in range(nsplit)]
    if two_dir:
        in_specs += [strip(j % 2) for j in range(nsplit)]
    in_specs += [hstrip] * nsplit
    in_specs += [s16(0), s16(1)]
    if two_dir:
        in_specs += [s16(0), s16(1)]
    in_specs += [full64]
    if two_dir:
        in_specs += [full64]
    in_specs += [full64, pl.BlockSpec((1, 64), lambda i: (0, 0))]
    return pl.pallas_call(
        kern, grid=(nb,), in_specs=in_specs,
        out_specs=[pl.BlockSpec((BR, 64), lambda i: (i, 0)),
                   pl.BlockSpec((2, 64), lambda i: (0, 0))],
        out_shape=[jax.ShapeDtypeStruct((n_pad, 64), jnp.float32),
                   jax.ShapeDtypeStruct((2, 64), jnp.float32)])


def _tc_bnrelu(n_nodes, nb, nsplit):
    width = 64 // nsplit

    def kern(*args):
        p, st, g, b = args[:4]
        outs = args[4:]
        mu = st[0:1, :] * (1.0 / n_nodes)
        var = st[1:2, :] * (1.0 / n_nodes) - mu * mu
        scale = g[...] * lax.rsqrt(var + 1e-5)
        h = jnp.maximum((p[...] - mu) * scale + b[...], 0.0)
        for j, o in enumerate(outs):
            o[...] = h[:, j * width:(j + 1) * width]

    n_pad = nb * BR
    return pl.pallas_call(
        kern, grid=(nb,),
        in_specs=[pl.BlockSpec((BR, 64), lambda i: (i, 0)),
                  pl.BlockSpec((2, 64), lambda i: (0, 0)),
                  pl.BlockSpec((1, 64), lambda i: (0, 0)),
                  pl.BlockSpec((1, 64), lambda i: (0, 0))],
        out_specs=[pl.BlockSpec((BR, width), lambda i: (i, 0))] * nsplit,
        out_shape=[jax.ShapeDtypeStruct((n_pad, width), jnp.float32)] * nsplit)


def _tc_bnrelumax(n_nodes, nb):
    def kern(p, st, g, b, emb_out):
        i = pl.program_id(0)
        mu = st[0:1, :] * (1.0 / n_nodes)
        var = st[1:2, :] * (1.0 / n_nodes) - mu * mu
        scale = g[...] * lax.rsqrt(var + 1e-5)
        h = jnp.maximum((p[...] - mu) * scale + b[...], 0.0)
        h = jnp.where(_rowmask(i, n_nodes), h, -jnp.inf)
        bm = jnp.max(h, 0, keepdims=True)

        @pl.when(i == 0)
        def _():
            emb_out[...] = bm

        @pl.when(i > 0)
        def _():
            emb_out[...] = jnp.maximum(emb_out[...], bm)

    return pl.pallas_call(
        kern, grid=(nb,),
        in_specs=[pl.BlockSpec((BR, 64), lambda i: (i, 0)),
                  pl.BlockSpec((2, 64), lambda i: (0, 0)),
                  pl.BlockSpec((1, 64), lambda i: (0, 0)),
                  pl.BlockSpec((1, 64), lambda i: (0, 0))],
        out_specs=pl.BlockSpec((1, 64), lambda i: (0, 0)),
        out_shape=jax.ShapeDtypeStruct((1, 64), jnp.float32))


def _tc_joint():
    def kern(de, re_, w, b, out):
        j = jnp.concatenate([de[...], re_[...]], axis=1)
        out[...] = jnp.maximum(
            jnp.dot(j, w[...], preferred_element_type=jnp.float32) + b[...], 0.0)

    return pl.pallas_call(kern, out_shape=jax.ShapeDtypeStruct((1, 128),
                                                               jnp.float32))


def _pack_edges(ei, e_real, nchunks_pad, n_nodes):
    npad = nchunks_pad * CH - e_real
    pad = n_nodes + (jnp.arange(npad, dtype=jnp.int32) % 128)
    src = jnp.concatenate([ei[0], pad]).reshape(nchunks_pad, CH)
    dst = jnp.concatenate([ei[1], pad]).reshape(nchunks_pad, CH)
    return jnp.stack([src, dst], axis=1)


def kernel(dag_x, dag_edge_index, res_x, res_edge_index, dag_f1_Wl, dag_f1_Wr,
           dag_f1_b, dag_b1_Wl, dag_b1_Wr, dag_b1_b, dag_f2_Wl, dag_f2_Wr,
           dag_f2_b, dag_b2_Wl, dag_b2_Wr, dag_b2_b, dag_bn1_g, dag_bn1_b,
           dag_bn2_g, dag_bn2_b, res_c1_Wl, res_c1_Wr, res_c1_b, res_c2_Wl,
           res_c2_Wr, res_c2_b, res_bn1_g, res_bn1_b, res_bn2_g, res_bn2_b,
           joint_W, joint_b):
    f32 = jnp.float32
    xd = jnp.zeros((NDP, 16), f32).at[:N_D, :5].set(dag_x).at[:N_D, 5].set(1.0)
    xr = jnp.zeros((NRP, 16), f32).at[:N_R, :2].set(res_x).at[:N_R, 2].set(1.0)
    e_d = _pack_edges(dag_edge_index, E_D, DCHP, N_D)
    e_r = _pack_edges(res_edge_index, E_R, RCHP, N_R)

    z16 = jnp.zeros((16, 64), f32)
    wl1f = z16.at[:5].set(dag_f1_Wl)
    wl1b = z16.at[:5].set(dag_b1_Wl)
    wr1 = z16.at[:5].set(dag_f1_Wr + dag_b1_Wr)
    b1 = (dag_f1_b + dag_b1_b).reshape(1, 64)
    wr2 = dag_f2_Wr + dag_b2_Wr
    b2 = (dag_f2_b + dag_b2_b).reshape(1, 64)
    rwl1 = z16.at[:2].set(res_c1_Wl)
    rwr1 = z16.at[:2].set(res_c1_Wr)

    sf = _sc_seg16(NDP, 25, True)(xd, e_d)
    sb = _sc_seg16(NDP, 25, False)(xd, e_d)
    p1, st1 = _tc_pre1(N_D, NB_D, 5, True)(sf, sf, sb, sb, xd, wl1f, wl1b,
                                           wr1, b1)
    h0, h1, h2, h3 = _tc_bnrelu(N_D, NB_D, 4)(p1, st1, dag_bn1_g.reshape(1, 64),
                                              dag_bn1_b.reshape(1, 64))
    s2fa = _sc_seg_strip(NDP, 50, True, 16)(h0, h1, e_d)
    s2fb = _sc_seg_strip(NDP, 50, True, 16)(h2, h3, e_d)
    s2ba = _sc_seg_strip(NDP, 50, False, 16)(h0, h1, e_d)
    s2bb = _sc_seg_strip(NDP, 50, False, 16)(h2, h3, e_d)
    p2, st2 = _tc_pre2(N_D, NB_D, 5, True, 4)(
        s2fa, s2fa, s2fb, s2fb, s2ba, s2ba, s2bb, s2bb,
        h0, h1, h2, h3, sf, sf, sb, sb,
        dag_f2_Wl, dag_b2_Wl, wr2, b2)
    demb = _tc_bnrelumax(N_D, NB_D)(p2, st2, dag_bn2_g.reshape(1, 64),
                                    dag_bn2_b.reshape(1, 64))

    rs = _sc_seg16(NRP, 10, True)(xr, e_r)
    q1, rt1 = _tc_pre1(N_R, NB_R, 2, False)(rs, rs, xr, rwl1, rwr1,
                                            res_c1_b.reshape(1, 64))
    gl, gr = _tc_bnrelu(N_R, NB_R, 2)(q1, rt1, res_bn1_g.reshape(1, 64),
                                      res_bn1_b.reshape(1, 64))
    rs2 = _sc_seg_strip(NRP, 20, True, 32)(gl, gr, e_r)
    q2, rt2 = _tc_pre2(N_R, NB_R, 2, False, 2)(
        rs2, rs2, gl, gr, rs, rs, res_c2_Wl, res_c2_Wr,
        res_c2_b.reshape(1, 64))
    remb = _tc_bnrelumax(N_R, NB_R)(q2, rt2, res_bn2_g.reshape(1, 64),
                                    res_bn2_b.reshape(1, 64))

    out = _tc_joint()(demb, remb, joint_W, joint_b.reshape(1, 128))
    return out.reshape(128)


# speedup vs baseline: 11.2016x; 1.1386x over previous
import jax
import jax.numpy as jnp
from jax import lax
from jax.experimental import pallas as pl
from jax.experimental.pallas import tpu as pltpu
from jax.experimental.pallas import tpu_sc as plsc

N_D, E_D = 50000, 800000
N_R, E_R = 10000, 320000
H = 64
BR = 1024
NB_D, NB_R = 49, 10
NDP = NB_D * BR
NRP = NB_R * BR
CH = 128
NTILES = 16
DCHP = 6400
RCHP = 2560

_mesh = lambda: plsc.VectorSubcoreMesh(core_axis_name="c", subcore_axis_name="s",
                                       num_cores=2, num_subcores=16)
_SC_PARAMS = pltpu.CompilerParams(use_tc_tiling_on_sc=False)


def _zero_fill(slab_v, srows):
    z = jnp.zeros((16,), jnp.float32)
    width = slab_v.shape[1]

    def zb(i, _):
        for k in range(8):
            for c0 in range(0, width, 16):
                slab_v[i * 8 + k, c0:c0 + 16] = z
        return _

    lax.fori_loop(0, srows // 8, zb, None)


def _sc_seg(n_pad, width, kb, bpt, phases, seg16, ntab):
    rows_per_tile = n_pad // NTILES
    srows = rows_per_tile // 2
    nph = len(phases)
    scratch = ([
        pltpu.VMEM((2, kb, 2, CH), jnp.int32),
        pltpu.VMEM((2, kb, CH, width), jnp.float32),
        pltpu.VMEM((srows, width), jnp.float32),
        pltpu.VMEM_SHARED((n_pad, width), jnp.float32),
    ] + [pltpu.SemaphoreType.DMA] * (kb + 2))

    def body(*args):
        tabs = args[:ntab]
        e_hbm = args[ntab]
        out = args[ntab + 1]
        idx_v, rows_v, slab_v, acc = args[ntab + 2:ntab + 6]
        sems = args[ntab + 6:]
        gsem, ssem = sems[:kb], sems[kb:]
        c = lax.axis_index("c")
        s = lax.axis_index("s")
        row0 = s * rows_per_tile

        for ph, (fwd, ia, ib) in enumerate(phases):
            gi, si = (0, 1) if fwd else (1, 0)
            dummy = tabs[ia]
            _zero_fill(slab_v, srows)
            for h in range(2):
                pltpu.sync_copy(slab_v, acc.at[pl.ds(row0 + h * srows, srows), :])
            plsc.subcore_barrier()
            if seg16:
                base = (s * 2 + c) * bpt * kb
            else:
                base = s * bpt * kb

            def blockwork(b, table):
                gd = [pltpu.async_copy(table.at[idx_v.at[b, k, gi]],
                                       rows_v.at[b, k], gsem[k])
                      for k in range(kb)]
                for k in range(kb):
                    gd[k].wait()
                    pltpu.async_copy(rows_v.at[b, k], acc.at[idx_v.at[b, k, si]],
                                     ssem[b], add=True)

            def pair(j, carry):
                for b in range(2):
                    t = j * 2 + b

                    @pl.when(t >= 2)
                    def _drain():
                        for k in range(kb):
                            pltpu.make_async_copy(dummy.at[pl.ds(0, CH)],
                                                  rows_v.at[b, k],
                                                  ssem[b]).wait()

                    pltpu.sync_copy(e_hbm.at[pl.ds(base + t * kb, kb)],
                                    idx_v.at[b])
                    if seg16:
                        blockwork(b, tabs[ia])
                    else:
                        @pl.when(c == 0)
                        def _g0():
                            blockwork(b, tabs[ia])

                        @pl.when(c == 1)
                        def _g1():
                            blockwork(b, tabs[ib])
                return carry

            lax.fori_loop(0, bpt // 2, pair, None)
            for b in range(2):
                for k in range(kb):
                    pltpu.make_async_copy(dummy.at[pl.ds(0, CH)],
                                          rows_v.at[b, k], ssem[b]).wait()
            plsc.subcore_barrier()
            off = (2 * ph + c) * n_pad + row0
            for h in range(2):
                pltpu.sync_copy(acc.at[pl.ds(row0 + h * srows, srows), :], slab_v)
                pltpu.sync_copy(slab_v, out.at[pl.ds(off + h * srows, srows), :])

    return pl.kernel(body,
                     out_type=jax.ShapeDtypeStruct((2 * nph * n_pad, width),
                                                   jnp.float32),
                     mesh=_mesh(), scratch_types=scratch,
                     compiler_params=_SC_PARAMS)


def _rowmask(i, n_nodes):
    rows = i * BR + lax.broadcasted_iota(jnp.int32, (BR, 1), 0)
    return rows < n_nodes


def _stats_accum(i, st_out, p):
    st = jnp.concatenate([jnp.sum(p, 0, keepdims=True),
                          jnp.sum(p * p, 0, keepdims=True)], 0)

    @pl.when(i == 0)
    def _():
        st_out[...] = st

    @pl.when(i > 0)
    def _():
        st_out[...] = st_out[...] + st


def _tc_pre1(n_nodes, nb, deg_col, two_dir):
    def kern(*args):
        if two_dir:
            (sf0, sf1, sb0, sb1, x, wlf, wlb, wr, b, p_out, st_out) = args
        else:
            (sf0, sf1, x, wlf, wr, b, p_out, st_out) = args
        i = pl.program_id(0)
        sfb = sf0[...] + sf1[...]
        mf = sfb / jnp.maximum(sfb[:, deg_col:deg_col + 1], 1.0)
        p = jnp.dot(mf, wlf[...], preferred_element_type=jnp.float32)
        if two_dir:
            sbb = sb0[...] + sb1[...]
            mb = sbb / jnp.maximum(sbb[:, deg_col:deg_col + 1], 1.0)
            p = p + jnp.dot(mb, wlb[...], preferred_element_type=jnp.float32)
        p = p + jnp.dot(x[...], wr[...], preferred_element_type=jnp.float32)
        p = p + b[...]
        p = jnp.where(_rowmask(i, n_nodes), p, 0.0)
        p_out[...] = p
        _stats_accum(i, st_out, p)

    n_pad = nb * BR
    half = lambda j: pl.BlockSpec((BR, 16), lambda i, j=j: (i + j * nb, 0))
    full16 = pl.BlockSpec((16, 64), lambda i: (0, 0))
    in_specs = [half(0), half(1)]
    if two_dir:
        in_specs += [half(2), half(3)]
    in_specs += [pl.BlockSpec((BR, 16), lambda i: (i, 0)), full16]
    if two_dir:
        in_specs += [full16]
    in_specs += [full16, pl.BlockSpec((1, 64), lambda i: (0, 0))]
    return pl.pallas_call(
        kern, grid=(nb,), in_specs=in_specs,
        out_specs=[pl.BlockSpec((BR, 64), lambda i: (i, 0)),
                   pl.BlockSpec((2, 64), lambda i: (0, 0))],
        out_shape=[jax.ShapeDtypeStruct((n_pad, 64), jnp.float32),
                   jax.ShapeDtypeStruct((2, 64), jnp.float32)])


def _tc_pre2(n_nodes, nb, deg_col, two_dir, nsplit):
    width = 64 // nsplit

    def kern(*args):
        args = list(args)
        s2f = [args.pop(0) for _ in range(nsplit)]
        s2b = [args.pop(0) for _ in range(nsplit)] if two_dir else None
        hs = [args.pop(0) for _ in range(nsplit)]
        sf0, sf1 = args.pop(0), args.pop(0)
        sb = (args.pop(0), args.pop(0)) if two_dir else None
        if two_dir:
            wlf, wlb, wr, b, p_out, st_out = args
        else:
            wlf, wr, b, p_out, st_out = args
        i = pl.program_id(0)
        degf = jnp.maximum(sf0[:, deg_col:deg_col + 1]
                           + sf1[:, deg_col:deg_col + 1], 1.0)
        m2f = jnp.concatenate([r[...] for r in s2f], axis=1) / degf
        p = jnp.dot(m2f, wlf[...], preferred_element_type=jnp.float32)
        if two_dir:
            degb = jnp.maximum(sb[0][:, deg_col:deg_col + 1]
                               + sb[1][:, deg_col:deg_col + 1], 1.0)
            m2b = jnp.concatenate([r[...] for r in s2b], axis=1) / degb
            p = p + jnp.dot(m2b, wlb[...], preferred_element_type=jnp.float32)
        hcat = jnp.concatenate([r[...] for r in hs], axis=1)
        p = p + jnp.dot(hcat, wr[...], preferred_element_type=jnp.float32)
        p = p + b[...]
        p = jnp.where(_rowmask(i, n_nodes), p, 0.0)
        p_out[...] = p
        _stats_accum(i, st_out, p)

    n_pad = nb * BR
    strip = lambda j: pl.BlockSpec((BR, width), lambda i, j=j: (i + j * nb, 0))
    s16 = lambda j: pl.BlockSpec((BR, 16), lambda i, j=j: (i + j * nb, 0))
    hstrip = pl.BlockSpec((BR, width), lambda i: (i, 0))
    full64 = pl.BlockSpec((64, 64), lambda i: (0, 0))
    in_specs = [strip(j) for j in range(nsplit)]
    if two_dir:
        in_specs += [strip(nsplit + j) for j in range(nsplit)]
    in_specs += [hstrip] * nsplit
    in_specs += [s16(0), s16(1)]
    if two_dir:
        in_specs += [s16(2), s16(3)]
    in_specs += [full64]
    if two_dir:
        in_specs += [full64]
    in_specs += [full64, pl.BlockSpec((1, 64), lambda i: (0, 0))]
    return pl.pallas_call(
        kern, grid=(nb,), in_specs=in_specs,
        out_specs=[pl.BlockSpec((BR, 64), lambda i: (i, 0)),
                   pl.BlockSpec((2, 64), lambda i: (0, 0))],
        out_shape=[jax.ShapeDtypeStruct((n_pad, 64), jnp.float32),
                   jax.ShapeDtypeStruct((2, 64), jnp.float32)])


def _tc_bnrelu(n_nodes, nb, nsplit):
    width = 64 // nsplit

    def kern(*args):
        p, st, g, b = args[:4]
        outs = args[4:]
        mu = st[0:1, :] * (1.0 / n_nodes)
        var = st[1:2, :] * (1.0 / n_nodes) - mu * mu
        scale = g[...] * lax.rsqrt(var + 1e-5)
        h = jnp.maximum((p[...] - mu) * scale + b[...], 0.0)
        for j, o in enumerate(outs):
            o[...] = h[:, j * width:(j + 1) * width]

    n_pad = nb * BR
    return pl.pallas_call(
        kern, grid=(nb,),
        in_specs=[pl.BlockSpec((BR, 64), lambda i: (i, 0)),
                  pl.BlockSpec((2, 64), lambda i: (0, 0)),
                  pl.BlockSpec((1, 64), lambda i: (0, 0)),
                  pl.BlockSpec((1, 64), lambda i: (0, 0))],
        out_specs=[pl.BlockSpec((BR, width), lambda i: (i, 0))] * nsplit,
        out_shape=[jax.ShapeDtypeStruct((n_pad, width), jnp.float32)] * nsplit)


def _tc_bnrelumax(n_nodes, nb):
    def kern(p, st, g, b, emb_out):
        i = pl.program_id(0)
        mu = st[0:1, :] * (1.0 / n_nodes)
        var = st[1:2, :] * (1.0 / n_nodes) - mu * mu
        scale = g[...] * lax.rsqrt(var + 1e-5)
        h = jnp.maximum((p[...] - mu) * scale + b[...], 0.0)
        h = jnp.where(_rowmask(i, n_nodes), h, -jnp.inf)
        bm = jnp.max(h, 0, keepdims=True)

        @pl.when(i == 0)
        def _():
            emb_out[...] = bm

        @pl.when(i > 0)
        def _():
            emb_out[...] = jnp.maximum(emb_out[...], bm)

    return pl.pallas_call(
        kern, grid=(nb,),
        in_specs=[pl.BlockSpec((BR, 64), lambda i: (i, 0)),
                  pl.BlockSpec((2, 64), lambda i: (0, 0)),
                  pl.BlockSpec((1, 64), lambda i: (0, 0)),
                  pl.BlockSpec((1, 64), lambda i: (0, 0))],
        out_specs=pl.BlockSpec((1, 64), lambda i: (0, 0)),
        out_shape=jax.ShapeDtypeStruct((1, 64), jnp.float32))


def _tc_joint():
    def kern(de, re_, w, b, out):
        j = jnp.concatenate([de[...], re_[...]], axis=1)
        out[...] = jnp.maximum(
            jnp.dot(j, w[...], preferred_element_type=jnp.float32) + b[...], 0.0)

    return pl.pallas_call(kern, out_shape=jax.ShapeDtypeStruct((1, 128),
                                                               jnp.float32))


def _pack_edges(ei, e_real, nchunks_pad, n_nodes):
    npad = nchunks_pad * CH - e_real
    pad = n_nodes + (jnp.arange(npad, dtype=jnp.int32) % 128)
    src = jnp.concatenate([ei[0], pad]).reshape(nchunks_pad, CH)
    dst = jnp.concatenate([ei[1], pad]).reshape(nchunks_pad, CH)
    return jnp.stack([src, dst], axis=1)


def kernel(dag_x, dag_edge_index, res_x, res_edge_index, dag_f1_Wl, dag_f1_Wr,
           dag_f1_b, dag_b1_Wl, dag_b1_Wr, dag_b1_b, dag_f2_Wl, dag_f2_Wr,
           dag_f2_b, dag_b2_Wl, dag_b2_Wr, dag_b2_b, dag_bn1_g, dag_bn1_b,
           dag_bn2_g, dag_bn2_b, res_c1_Wl, res_c1_Wr, res_c1_b, res_c2_Wl,
           res_c2_Wr, res_c2_b, res_bn1_g, res_bn1_b, res_bn2_g, res_bn2_b,
           joint_W, joint_b):
    f32 = jnp.float32
    xd = jnp.zeros((NDP, 16), f32).at[:N_D, :5].set(dag_x).at[:N_D, 5].set(1.0)
    xr = jnp.zeros((NRP, 16), f32).at[:N_R, :2].set(res_x).at[:N_R, 2].set(1.0)
    e_d = _pack_edges(dag_edge_index, E_D, DCHP, N_D)
    e_r = _pack_edges(res_edge_index, E_R, RCHP, N_R)

    z16 = jnp.zeros((16, 64), f32)
    wl1f = z16.at[:5].set(dag_f1_Wl)
    wl1b = z16.at[:5].set(dag_b1_Wl)
    wr1 = z16.at[:5].set(dag_f1_Wr + dag_b1_Wr)
    b1 = (dag_f1_b + dag_b1_b).reshape(1, 64)
    wr2 = dag_f2_Wr + dag_b2_Wr
    b2 = (dag_f2_b + dag_b2_b).reshape(1, 64)
    rwl1 = z16.at[:2].set(res_c1_Wl)
    rwr1 = z16.at[:2].set(res_c1_Wr)

    s1 = _sc_seg(NDP, 16, 10, 20, [(True, 0, 0), (False, 0, 0)], True, 1)(
        xd, e_d)
    p1, st1 = _tc_pre1(N_D, NB_D, 5, True)(s1, s1, s1, s1, xd, wl1f, wl1b,
                                           wr1, b1)
    h0, h1, h2, h3 = _tc_bnrelu(N_D, NB_D, 4)(p1, st1, dag_bn1_g.reshape(1, 64),
                                              dag_bn1_b.reshape(1, 64))
    s2 = _sc_seg(NDP, 16, 8, 50,
                 [(True, 0, 1), (True, 2, 3), (False, 0, 1), (False, 2, 3)],
                 False, 4)(h0, h1, h2, h3, e_d)
    p2, st2 = _tc_pre2(N_D, NB_D, 5, True, 4)(
        s2, s2, s2, s2, s2, s2, s2, s2,
        h0, h1, h2, h3, s1, s1, s1, s1,
        dag_f2_Wl, dag_b2_Wl, wr2, b2)
    demb = _tc_bnrelumax(N_D, NB_D)(p2, st2, dag_bn2_g.reshape(1, 64),
                                    dag_bn2_b.reshape(1, 64))

    rs = _sc_seg(NRP, 16, 8, 10, [(True, 0, 0)], True, 1)(xr, e_r)
    q1, rt1 = _tc_pre1(N_R, NB_R, 2, False)(rs, rs, xr, rwl1, rwr1,
                                            res_c1_b.reshape(1, 64))
    gl, gr = _tc_bnrelu(N_R, NB_R, 2)(q1, rt1, res_bn1_g.reshape(1, 64),
                                      res_bn1_b.reshape(1, 64))
    rs2 = _sc_seg(NRP, 32, 8, 20, [(True, 0, 1)], False, 2)(gl, gr, e_r)
    q2, rt2 = _tc_pre2(N_R, NB_R, 2, False, 2)(
        rs2, rs2, gl, gr, rs, rs, res_c2_Wl, res_c2_Wr,
        res_c2_b.reshape(1, 64))
    remb = _tc_bnrelumax(N_R, NB_R)(q2, rt2, res_bn2_g.reshape(1, 64),
                                    res_bn2_b.reshape(1, 64))

    out = _tc_joint()(demb, remb, joint_W, joint_b.reshape(1, 128))
    return out.reshape(128)
